# baseline (speedup 1.0000x reference)
import jax
import jax.numpy as jnp
from jax import lax
from jax.experimental import pallas as pl
from jax.experimental.pallas import tpu as pltpu

LANE = 128
GATES = 3
U = 8
FEAT = 44
FEAT_PAD = 128
OUT_COLS = 25
HV, HN = 24, 48
HVN = HV + HN


def _shift_gate_cols(w, h, s):
    r = w.shape[0]
    w3 = w.reshape(r, GATES, LANE)
    return jnp.pad(w3[:, :, :h], ((0, 0), (0, 0), (s, LANE - s - h))).reshape(r, GATES * LANE)


def _halve_rz(w):
    r = w.shape[0]
    w3 = w.reshape(r, GATES, LANE)
    return jnp.concatenate([w3[:, :2] * 0.5, w3[:, 2:]], axis=1).reshape(r, GATES * LANE)


def _to216(w):
    r = w.shape[0]
    return w.reshape(r, GATES, LANE)[:, :, :HVN].reshape(r, GATES * HVN)


def _stack_tiles(wh, wx):
    t0 = jnp.concatenate([wh[:, :2 * LANE], wx[:, :2 * LANE]], axis=0)
    t1 = jnp.concatenate([wh[:, 2 * LANE:], wx[:, 2 * LANE:]], axis=0)
    t1 = jnp.pad(t1, ((0, 0), (0, LANE)))
    return t0.astype(jnp.bfloat16), t1.astype(jnp.bfloat16)


def _rnnoise_kernel(
    xp_v_ref, xp_n_ref, xp_d_ref,
    vn_t_ref, bhn_vn_ref,
    d_t0_ref, d_t1_ref, bhn_d_ref,
    vng_ref, deng_ref,
    hvn_s, hd_s,
):
    tq = xp_v_ref.shape[0]
    nb = tq // U

    @pl.when(pl.program_id(0) == 0)
    def _init():
        hvn_s[...] = jnp.zeros_like(hvn_s)
        hd_s[...] = jnp.zeros_like(hd_s)

    @pl.when(pl.program_id(0) == 0)
    def _drain_mrb():
        junk = jnp.zeros((1, 1), jnp.float32)
        for mxu in (0, 1):
            for base in (0, 128):
                j = pltpu.matmul_pop(acc_addr=base, shape=(512, 256),
                                     dtype=jnp.float32, mxu_index=mxu)
                junk = junk + j[:1, :1]
        hvn_s[0:1, 0:1] = hvn_s[0:1, 0:1] + junk * 0.0

    pltpu.matmul_push_rhs(vn_t_ref[...], staging_register=0, mxu_index=0)
    pltpu.matmul_acc_lhs(acc_addr=56, lhs=jnp.zeros((16, 2 * LANE), jnp.bfloat16),
                         mxu_index=0, load_staged_rhs=0)

    bhn_vn = bhn_vn_ref[...][:, :HVN]
    bhn_d = bhn_d_ref[...]

    vmask1 = lax.broadcasted_iota(jnp.int32, (1, HVN), 1) < HV
    vmask8 = lax.broadcasted_iota(jnp.int32, (U, HVN), 1) < HV

    def _base(b):
        if isinstance(b, int):
            return b * U
        return pl.multiple_of(b * U, U)

    def _expand216(m):
        return jnp.concatenate(
            [jnp.pad(m[:, g * HVN:(g + 1) * HVN], ((0, 0), (0, LANE - HVN)))
             for g in range(GATES)], axis=1)

    def vn_step_mm(lhs):
        pltpu.matmul_acc_lhs(acc_addr=0, lhs=lhs, mxu_index=0, load_staged_rhs=None)
        return pltpu.matmul_pop(acc_addr=0, shape=(8, 2 * LANE), dtype=jnp.float32,
                                mxu_index=0)[0:1, :]

    def vn_producer_mm(prod8):
        lhs = jnp.pad(prod8, ((0, 16 - U), (LANE, LANE - HVN)))
        pltpu.matmul_acc_lhs(acc_addr=32, lhs=lhs, mxu_index=0, load_staged_rhs=None)
        g = pltpu.matmul_pop(acc_addr=32, shape=(U, 2 * LANE), dtype=jnp.float32,
                             mxu_index=0)
        return _expand216(g)

    def d_push():
        pltpu.matmul_push_rhs(d_t0_ref[...], staging_register=0, mxu_index=1)
        pltpu.matmul_push_rhs(d_t1_ref[...], staging_register=1, mxu_index=1)

    def d_step_mm(lhs):
        d_push()
        pltpu.matmul_acc_lhs(acc_addr=0, lhs=lhs, mxu_index=1, load_staged_rhs=0)
        pltpu.matmul_acc_lhs(acc_addr=16, lhs=lhs, mxu_index=1, load_staged_rhs=1)
        hm_rz = pltpu.matmul_pop(acc_addr=0, shape=(8, 2 * LANE), dtype=jnp.float32,
                                 mxu_index=1)[0:1, :]
        hm_n = pltpu.matmul_pop(acc_addr=16, shape=(8, 2 * LANE), dtype=jnp.float32,
                                mxu_index=1)[0:1, :LANE]
        return hm_rz, hm_n

    def d_producer_mm(prod8):
        d_push()
        lhs = jnp.pad(prod8, ((0, 16 - U), (LANE, 0)))
        pltpu.matmul_acc_lhs(acc_addr=32, lhs=lhs, mxu_index=1, load_staged_rhs=0)
        pltpu.matmul_acc_lhs(acc_addr=48, lhs=lhs, mxu_index=1, load_staged_rhs=1)
        g_rz = pltpu.matmul_pop(acc_addr=32, shape=(U, 2 * LANE), dtype=jnp.float32,
                                mxu_index=1)
        g_n = pltpu.matmul_pop(acc_addr=48, shape=(U, 2 * LANE), dtype=jnp.float32,
                               mxu_index=1)[:, :LANE]
        return jnp.concatenate([g_rz, g_n], axis=1)

    def vn_gru_steps(xp_blk, h):
        xrz = jnp.concatenate([xp_blk[:, :HVN], xp_blk[:, LANE:LANE + HVN]], axis=1)
        xn = xp_blk[:, 2 * LANE:2 * LANE + HVN]
        rows = []
        for u in range(U):
            lhs = jnp.pad(h.astype(jnp.bfloat16), ((0, 15), (0, 2 * LANE - HVN)))
            hm = vn_step_mm(lhs)
            th = jnp.tanh(xrz[u:u + 1, :] + hm[:, :2 * HVN])
            z = 0.5 * th[:, HVN:2 * HVN] + 0.5
            p = 0.5 * (hm[:, 2 * HVN:GATES * HVN] + bhn_vn)
            n = jnp.tanh(xn[u:u + 1, :] + p + th[:, :HVN] * p)
            h = n + z * (h - n)
            rows.append(h)
        return jnp.concatenate(rows, axis=0), h

    def d_gru_steps(xp_blk, h):
        rows = []
        for u in range(U):
            xrow = xp_blk[u:u + 1, :]
            lhs = jnp.pad(h.astype(jnp.bfloat16), ((0, 15), (0, LANE)))
            hm_rz, hm_n = d_step_mm(lhs)
            th = jnp.tanh(xrow[:, :2 * LANE] + hm_rz)
            th_r, th_z = th[:, :LANE], th[:, LANE:]
            z = 0.5 * th_z + 0.5
            p = 0.5 * (hm_n + bhn_d)
            n = jnp.tanh(xrow[:, 2 * LANE:] + p + th_r * p)
            h = n + z * (h - n)
            rows.append(h)
        return jnp.concatenate(rows, axis=0), h

    def wave(bm, with_noise, bd, prev):
        if bm is not None:
            if bm == "noise_only":
                nbase = _base(nb - 1)
                xp = xp_n_ref[pl.ds(nbase, U), :].astype(jnp.float32) + vn_producer_mm(prev.astype(jnp.bfloat16))
            else:
                vbase = _base(bm)
                xp = xp_v_ref[pl.ds(vbase, U), :].astype(jnp.float32)
                if with_noise:
                    nbase = _base(bm - 1)
                    xp = (xp + xp_n_ref[pl.ds(nbase, U), :].astype(jnp.float32)
                          + vn_producer_mm(prev.astype(jnp.bfloat16)))
            h0 = hvn_s[...]
        if bd is not None:
            dbase = _base(bd)
            xp_d = (xp_d_ref[pl.ds(dbase, U), :].astype(jnp.float32)
                    + d_producer_mm(vng_ref[pl.ds(dbase, U), :].astype(jnp.bfloat16)))
            hd0 = hd_s[...]
        if bm is not None:
            hrows, h1 = vn_gru_steps(xp, h0)
            act = jnp.where(vmask8, jnp.tanh(hrows), jnp.maximum(hrows, 0.0))
        if bd is not None:
            drows, hd1 = d_gru_steps(xp_d, hd0)
            d_out = jnp.tanh(drows)
        new_prev = prev
        if bm is not None:
            if with_noise:
                vng_ref[pl.ds(nbase, U), :] = jnp.pad(jnp.where(vmask8, prev, act),
                                                      ((0, 0), (0, LANE - HVN)))
            hvn_s[...] = h1
            new_prev = act
        if bd is not None:
            deng_ref[pl.ds(dbase, U), :] = d_out
            hd_s[...] = hd1
        return new_prev

    prev = wave(0, False, None, jnp.zeros((U, HVN), jnp.float32))
    hvn_s[...] = jnp.where(vmask1, hvn_s[...], 0.0)
    prev = wave(1, True, None, prev)

    def body(b, prev):
        return wave(b, True, b - 2, prev)

    prev = lax.fori_loop(2, nb, body, prev)

    h_keep = hvn_s[...]
    prev = wave("noise_only", True, nb - 2, prev)
    hvn_s[...] = jnp.where(vmask1, h_keep, hvn_s[...])
    wave(None, False, nb - 1, prev)


def _precompute_kernel(x_ref, wd_ref, bd_ref, wi_v_ref, bi_v_ref,
                       wi_nx_ref, wi_nd_ref, bi_n_ref, wi_dx_ref, bi_d_ref,
                       xp_v_ref, xp_n_ref, xp_d_ref):
    x = x_ref[...]
    x_b = x.astype(jnp.bfloat16)
    dense = jnp.tanh(jnp.dot(x, wd_ref[...], preferred_element_type=jnp.float32)
                     + bd_ref[...])
    dense_b = dense.astype(jnp.bfloat16)
    mm = lambda a, b: jnp.dot(a, b, preferred_element_type=jnp.float32)
    xp_v_ref[...] = (mm(dense_b, wi_v_ref[...]) + bi_v_ref[...]).astype(jnp.bfloat16)
    xp_n_ref[...] = (mm(x_b, wi_nx_ref[...]) + mm(dense_b, wi_nd_ref[...])
                     + bi_n_ref[...]).astype(jnp.bfloat16)
    xp_d_ref[...] = (mm(x_b, wi_dx_ref[...]) + bi_d_ref[...]).astype(jnp.bfloat16)


def _output_kernel(vng_ref, deng_ref, wo_v_ref, wo_d_ref, bo_ref, out_ref):
    mm = lambda a, b: jnp.dot(a, b, preferred_element_type=jnp.float32)
    out_ref[...] = jax.nn.sigmoid(
        mm(vng_ref[...], wo_v_ref[...]) + mm(deng_ref[...], wo_d_ref[...]) + bo_ref[...])


def _forward(x, pre_params, rec_params, out_params, *, tq=512):
    T, F = x.shape
    t_pad = ((T + tq - 1) // tq) * tq
    x = jnp.pad(x.astype(jnp.float32), ((0, t_pad - T), (0, FEAT_PAD - F)))
    g = t_pad // tq

    def _full(p):
        return pl.BlockSpec(p.shape, lambda i: (0, 0))

    blk = lambda w: pl.BlockSpec((tq, w), lambda i: (i, 0))
    xpw = GATES * LANE

    xp_v, xp_n, xp_d = pl.pallas_call(
        _precompute_kernel,
        out_shape=[jax.ShapeDtypeStruct((t_pad, xpw), jnp.bfloat16)] * 3,
        grid=(g,),
        in_specs=[blk(FEAT_PAD)] + [_full(p) for p in pre_params],
        out_specs=[blk(xpw)] * 3,
        compiler_params=pltpu.CompilerParams(
            dimension_semantics=("parallel",),
            vmem_limit_bytes=100 * 1024 * 1024,
        ),
    )(x, *pre_params)

    vng, deng = pl.pallas_call(
        _rnnoise_kernel,
        out_shape=[jax.ShapeDtypeStruct((t_pad, LANE), jnp.float32)] * 2,
        grid=(g,),
        in_specs=[blk(xpw)] * 3 + [_full(p) for p in rec_params],
        out_specs=[blk(LANE)] * 2,
        scratch_shapes=[
            pltpu.VMEM((1, HVN), jnp.float32),
            pltpu.VMEM((1, LANE), jnp.float32),
        ],
        compiler_params=pltpu.CompilerParams(
            dimension_semantics=("arbitrary",),
            vmem_limit_bytes=100 * 1024 * 1024,
        ),
    )(xp_v, xp_n, xp_d, *rec_params)

    out = pl.pallas_call(
        _output_kernel,
        out_shape=jax.ShapeDtypeStruct((t_pad, LANE), jnp.float32),
        grid=(g,),
        in_specs=[blk(LANE)] * 2 + [_full(p) for p in out_params],
        out_specs=blk(LANE),
        compiler_params=pltpu.CompilerParams(
            dimension_semantics=("parallel",),
            vmem_limit_bytes=100 * 1024 * 1024,
        ),
    )(vng, deng, *out_params)
    return out[:T, 0:1], out[:T, 1:OUT_COLS]


def _repack(wd, bd, wi_v, bi_v, wh_v, bhn_v, wi_nx, wi_nd, wi_nv, bi_n,
            wh_n, bhn_n, wi_dx, wi_dv, wi_dn, bi_d, wh_d, bhn_d, wo_v, wo_d, bo):
    wi_nx_s = _shift_gate_cols(wi_nx, HN, HV)
    wi_nd_s = _shift_gate_cols(wi_nd, HN, HV)
    wi_nv_s = _shift_gate_cols(wi_nv, HN, HV)
    bi_n_s = _shift_gate_cols(bi_n, HN, HV)
    wh_n_s = jnp.pad(_shift_gate_cols(wh_n, HN, HV)[:HN], ((HV, LANE - HVN), (0, 0)))
    wh_vn = _halve_rz(wh_v + wh_n_s)
    bhn_vn = bhn_v + jnp.pad(bhn_n[:, :HN], ((0, 0), (HV, LANE - HVN)))
    wi_dvn = _halve_rz(wi_dv + jnp.pad(wi_dn[:HN], ((HV, LANE - HVN), (0, 0))))
    vn_t = jnp.pad(jnp.concatenate(
        [_to216(wh_vn), _to216(_halve_rz(wi_nv_s))], axis=0),
        ((0, 0), (0, 2 * LANE - GATES * HVN))).astype(jnp.bfloat16)
    d_t0, d_t1 = _stack_tiles(_halve_rz(wh_d), wi_dvn)
    pre_params = (wd, bd, _halve_rz(wi_v), _halve_rz(bi_v),
                  _halve_rz(wi_nx_s), _halve_rz(wi_nd_s), _halve_rz(bi_n_s),
                  _halve_rz(wi_dx), _halve_rz(bi_d))
    rec_params = (vn_t, bhn_vn, d_t0, d_t1, bhn_d)
    out_params = (wo_v, wo_d, bo)
    return pre_params, rec_params, out_params


def kernel(x, wd, bd, wi_v, bi_v, wh_v, bhn_v, wi_nx, wi_nd, wi_nv, bi_n,
           wh_n, bhn_n, wi_dx, wi_dv, wi_dn, bi_d, wh_d, bhn_d, wo_v, wo_d, bo):
    pre_params, rec_params, out_params = _repack(
        wd, bd, wi_v, bi_v, wh_v, bhn_v, wi_nx, wi_nd, wi_nv, bi_n,
        wh_n, bhn_n, wi_dx, wi_dv, wi_dn, bi_d, wh_d, bhn_d, wo_v, wo_d, bo)
    return _forward(x, pre_params, rec_params, out_params, tq=1024)


# speedup vs baseline: 1.6940x; 1.6940x over previous
import jax
import jax.numpy as jnp
from jax import lax
from jax.experimental import pallas as pl
from jax.experimental.pallas import tpu as pltpu

LANE = 128
GATES = 3
U = 8
FEAT = 44
FEAT_PAD = 128
OUT_COLS = 25
HV, HN = 24, 48
HVN = HV + HN


def _shift_gate_cols(w, h, s):
    r = w.shape[0]
    w3 = w.reshape(r, GATES, LANE)
    return jnp.pad(w3[:, :, :h], ((0, 0), (0, 0), (s, LANE - s - h))).reshape(r, GATES * LANE)


def _halve_rz(w):
    r = w.shape[0]
    w3 = w.reshape(r, GATES, LANE)
    return jnp.concatenate([w3[:, :2] * 0.5, w3[:, 2:]], axis=1).reshape(r, GATES * LANE)


def _to216(w):
    r = w.shape[0]
    return w.reshape(r, GATES, LANE)[:, :, :HVN].reshape(r, GATES * HVN)


def _stack_tiles(wh, wx):
    t0 = jnp.concatenate([wh[:, :2 * LANE], wx[:, :2 * LANE]], axis=0)
    t1 = jnp.concatenate([wh[:, 2 * LANE:], wx[:, 2 * LANE:]], axis=0)
    t1 = jnp.pad(t1, ((0, 0), (0, LANE)))
    return t0.astype(jnp.bfloat16), t1.astype(jnp.bfloat16)


def _rnnoise_kernel(
    xp_v_ref, xp_n_ref, xp_d_ref,
    vn_t_ref, bhn_vn_ref,
    d_t0_ref, d_t1_ref, bhn_d_ref,
    vng_ref, deng_ref,
    hvn_s, hd_s,
):
    tq = xp_v_ref.shape[0]
    nb = tq // U

    @pl.when(pl.program_id(0) == 0)
    def _init():
        hvn_s[...] = jnp.zeros_like(hvn_s)
        hd_s[...] = jnp.zeros_like(hd_s)

    @pl.when(pl.program_id(0) == 0)
    def _drain_mrb():
        junk = jnp.zeros((1, 1), jnp.float32)
        for mxu in (0, 1):
            for base in (0, 128):
                j = pltpu.matmul_pop(acc_addr=base, shape=(512, 256),
                                     dtype=jnp.float32, mxu_index=mxu)
                junk = junk + j[:1, :1]
        hvn_s[0:1, 0:1] = hvn_s[0:1, 0:1] + junk * 0.0

    pltpu.matmul_push_rhs(vn_t_ref[...], staging_register=0, mxu_index=0)
    pltpu.matmul_acc_lhs(acc_addr=56, lhs=jnp.zeros((16, 2 * LANE), jnp.bfloat16),
                         mxu_index=0, load_staged_rhs=0)

    bhn_vn = bhn_vn_ref[...]
    bhn_d = bhn_d_ref[...]

    vmask1 = lax.broadcasted_iota(jnp.int32, (1, LANE), 1) < HV
    vmask8 = lax.broadcasted_iota(jnp.int32, (U, LANE), 1) < HV

    def _base(b):
        if isinstance(b, int):
            return b * U
        return pl.multiple_of(b * U, U)

    def _expand216(m):
        return jnp.concatenate(
            [jnp.pad(m[:, g * HVN:(g + 1) * HVN], ((0, 0), (0, LANE - HVN)))
             for g in range(GATES)], axis=1)

    def vn_step_mm(lhs):
        pltpu.matmul_acc_lhs(acc_addr=0, lhs=lhs, mxu_index=0, load_staged_rhs=None)
        hm = pltpu.matmul_pop(acc_addr=0, shape=(8, 2 * LANE), dtype=jnp.float32,
                              mxu_index=0)[0:1, :]
        hm = _expand216(hm)
        return hm[:, :2 * LANE], hm[:, 2 * LANE:]

    def vn_producer_mm(prod8):
        lhs = jnp.pad(prod8, ((0, 16 - U), (LANE, 0)))
        pltpu.matmul_acc_lhs(acc_addr=32, lhs=lhs, mxu_index=0, load_staged_rhs=None)
        g = pltpu.matmul_pop(acc_addr=32, shape=(U, 2 * LANE), dtype=jnp.float32,
                             mxu_index=0)
        return _expand216(g)

    def d_push():
        pltpu.matmul_push_rhs(d_t0_ref[...], staging_register=0, mxu_index=1)
        pltpu.matmul_push_rhs(d_t1_ref[...], staging_register=1, mxu_index=1)

    def d_step_mm(lhs):
        d_push()
        pltpu.matmul_acc_lhs(acc_addr=0, lhs=lhs, mxu_index=1, load_staged_rhs=0)
        pltpu.matmul_acc_lhs(acc_addr=16, lhs=lhs, mxu_index=1, load_staged_rhs=1)
        hm_rz = pltpu.matmul_pop(acc_addr=0, shape=(8, 2 * LANE), dtype=jnp.float32,
                                 mxu_index=1)[0:1, :]
        hm_n = pltpu.matmul_pop(acc_addr=16, shape=(8, 2 * LANE), dtype=jnp.float32,
                                mxu_index=1)[0:1, :LANE]
        return hm_rz, hm_n

    def d_producer_mm(prod8):
        d_push()
        lhs = jnp.pad(prod8, ((0, 16 - U), (LANE, 0)))
        pltpu.matmul_acc_lhs(acc_addr=32, lhs=lhs, mxu_index=1, load_staged_rhs=0)
        pltpu.matmul_acc_lhs(acc_addr=48, lhs=lhs, mxu_index=1, load_staged_rhs=1)
        g_rz = pltpu.matmul_pop(acc_addr=32, shape=(U, 2 * LANE), dtype=jnp.float32,
                                mxu_index=1)
        g_n = pltpu.matmul_pop(acc_addr=48, shape=(U, 2 * LANE), dtype=jnp.float32,
                               mxu_index=1)[:, :LANE]
        return jnp.concatenate([g_rz, g_n], axis=1)

    def gru_steps(xp_blk, bhn, h, mxu):
        rows = []
        for u in range(U):
            xrow = xp_blk[u:u + 1, :]
            lhs = jnp.pad(h.astype(jnp.bfloat16), ((0, 15), (0, LANE)))
            hm_rz, hm_n = vn_step_mm(lhs) if mxu == 0 else d_step_mm(lhs)
            th = jnp.tanh(xrow[:, :2 * LANE] + hm_rz)
            th_r, th_z = th[:, :LANE], th[:, LANE:]
            z = 0.5 * th_z + 0.5
            p = 0.5 * (hm_n + bhn)
            n = jnp.tanh(xrow[:, 2 * LANE:] + p + th_r * p)
            h = n + z * (h - n)
            rows.append(h)
        return jnp.concatenate(rows, axis=0), h

    def wave(bm, with_noise, bd, prev):
        if bm is not None:
            if bm == "noise_only":
                nbase = _base(nb - 1)
                xp = xp_n_ref[pl.ds(nbase, U), :].astype(jnp.float32) + vn_producer_mm(prev.astype(jnp.bfloat16))
            else:
                vbase = _base(bm)
                xp = xp_v_ref[pl.ds(vbase, U), :].astype(jnp.float32)
                if with_noise:
                    nbase = _base(bm - 1)
                    xp = (xp + xp_n_ref[pl.ds(nbase, U), :].astype(jnp.float32)
                          + vn_producer_mm(prev.astype(jnp.bfloat16)))
            h0 = hvn_s[...]
        if bd is not None:
            dbase = _base(bd)
            xp_d = (xp_d_ref[pl.ds(dbase, U), :].astype(jnp.float32)
                    + d_producer_mm(vng_ref[pl.ds(dbase, U), :].astype(jnp.bfloat16)))
            hd0 = hd_s[...]
        if bm is not None:
            hrows, h1 = gru_steps(xp, bhn_vn, h0, 0)
            act = jnp.where(vmask8, jnp.tanh(hrows), jnp.maximum(hrows, 0.0))
        if bd is not None:
            drows, hd1 = gru_steps(xp_d, bhn_d, hd0, 1)
            d_out = jnp.tanh(drows)
        new_prev = prev
        if bm is not None:
            if with_noise:
                vng_ref[pl.ds(nbase, U), :] = jnp.where(vmask8, prev, act)
            hvn_s[...] = h1
            new_prev = act
        if bd is not None:
            deng_ref[pl.ds(dbase, U), :] = d_out
            hd_s[...] = hd1
        return new_prev

    prev = wave(0, False, None, jnp.zeros((U, LANE), jnp.float32))
    hvn_s[...] = jnp.where(vmask1, hvn_s[...], 0.0)
    prev = wave(1, True, None, prev)

    def body(b, prev):
        return wave(b, True, b - 2, prev)

    prev = lax.fori_loop(2, nb, body, prev)

    h_keep = hvn_s[...]
    prev = wave("noise_only", True, nb - 2, prev)
    hvn_s[...] = jnp.where(vmask1, h_keep, hvn_s[...])
    wave(None, False, nb - 1, prev)


def _precompute_kernel(x_ref, wd_ref, bd_ref, wi_v_ref, bi_v_ref,
                       wi_nx_ref, wi_nd_ref, bi_n_ref, wi_dx_ref, bi_d_ref,
                       xp_v_ref, xp_n_ref, xp_d_ref):
    x = x_ref[...]
    x_b = x.astype(jnp.bfloat16)
    dense = jnp.tanh(jnp.dot(x, wd_ref[...], preferred_element_type=jnp.float32)
                     + bd_ref[...])
    dense_b = dense.astype(jnp.bfloat16)
    mm = lambda a, b: jnp.dot(a, b, preferred_element_type=jnp.float32)
    xp_v_ref[...] = (mm(dense_b, wi_v_ref[...]) + bi_v_ref[...]).astype(jnp.bfloat16)
    xp_n_ref[...] = (mm(x_b, wi_nx_ref[...]) + mm(dense_b, wi_nd_ref[...])
                     + bi_n_ref[...]).astype(jnp.bfloat16)
    xp_d_ref[...] = (mm(x_b, wi_dx_ref[...]) + bi_d_ref[...]).astype(jnp.bfloat16)


def _output_kernel(vng_ref, deng_ref, wo_v_ref, wo_d_ref, bo_ref, out_ref):
    mm = lambda a, b: jnp.dot(a, b, preferred_element_type=jnp.float32)
    out_ref[...] = jax.nn.sigmoid(
        mm(vng_ref[...], wo_v_ref[...]) + mm(deng_ref[...], wo_d_ref[...]) + bo_ref[...])


def _forward(x, pre_params, rec_params, out_params, *, tq=512):
    T, F = x.shape
    t_pad = ((T + tq - 1) // tq) * tq
    x = jnp.pad(x.astype(jnp.float32), ((0, t_pad - T), (0, FEAT_PAD - F)))
    g = t_pad // tq

    def _full(p):
        return pl.BlockSpec(p.shape, lambda i: (0, 0))

    blk = lambda w: pl.BlockSpec((tq, w), lambda i: (i, 0))
    xpw = GATES * LANE

    xp_v, xp_n, xp_d = pl.pallas_call(
        _precompute_kernel,
        out_shape=[jax.ShapeDtypeStruct((t_pad, xpw), jnp.bfloat16)] * 3,
        grid=(g,),
        in_specs=[blk(FEAT_PAD)] + [_full(p) for p in pre_params],
        out_specs=[blk(xpw)] * 3,
        compiler_params=pltpu.CompilerParams(
            dimension_semantics=("parallel",),
            vmem_limit_bytes=100 * 1024 * 1024,
        ),
    )(x, *pre_params)

    vng, deng = pl.pallas_call(
        _rnnoise_kernel,
        out_shape=[jax.ShapeDtypeStruct((t_pad, LANE), jnp.float32)] * 2,
        grid=(g,),
        in_specs=[blk(xpw)] * 3 + [_full(p) for p in rec_params],
        out_specs=[blk(LANE)] * 2,
        scratch_shapes=[
            pltpu.VMEM((1, LANE), jnp.float32),
            pltpu.VMEM((1, LANE), jnp.float32),
        ],
        compiler_params=pltpu.CompilerParams(
            dimension_semantics=("arbitrary",),
            vmem_limit_bytes=100 * 1024 * 1024,
        ),
    )(xp_v, xp_n, xp_d, *rec_params)

    out = pl.pallas_call(
        _output_kernel,
        out_shape=jax.ShapeDtypeStruct((t_pad, LANE), jnp.float32),
        grid=(g,),
        in_specs=[blk(LANE)] * 2 + [_full(p) for p in out_params],
        out_specs=blk(LANE),
        compiler_params=pltpu.CompilerParams(
            dimension_semantics=("parallel",),
            vmem_limit_bytes=100 * 1024 * 1024,
        ),
    )(vng, deng, *out_params)
    return out[:T, 0:1], out[:T, 1:OUT_COLS]


def _repack(wd, bd, wi_v, bi_v, wh_v, bhn_v, wi_nx, wi_nd, wi_nv, bi_n,
            wh_n, bhn_n, wi_dx, wi_dv, wi_dn, bi_d, wh_d, bhn_d, wo_v, wo_d, bo):
    wi_nx_s = _shift_gate_cols(wi_nx, HN, HV)
    wi_nd_s = _shift_gate_cols(wi_nd, HN, HV)
    wi_nv_s = _shift_gate_cols(wi_nv, HN, HV)
    bi_n_s = _shift_gate_cols(bi_n, HN, HV)
    wh_n_s = jnp.pad(_shift_gate_cols(wh_n, HN, HV)[:HN], ((HV, LANE - HVN), (0, 0)))
    wh_vn = _halve_rz(wh_v + wh_n_s)
    bhn_vn = bhn_v + jnp.pad(bhn_n[:, :HN], ((0, 0), (HV, LANE - HVN)))
    wi_dvn = _halve_rz(wi_dv + jnp.pad(wi_dn[:HN], ((HV, LANE - HVN), (0, 0))))
    vn_t = jnp.pad(jnp.concatenate(
        [_to216(wh_vn), _to216(_halve_rz(wi_nv_s))], axis=0),
        ((0, 0), (0, 2 * LANE - GATES * HVN))).astype(jnp.bfloat16)
    d_t0, d_t1 = _stack_tiles(_halve_rz(wh_d), wi_dvn)
    pre_params = (wd, bd, _halve_rz(wi_v), _halve_rz(bi_v),
                  _halve_rz(wi_nx_s), _halve_rz(wi_nd_s), _halve_rz(bi_n_s),
                  _halve_rz(wi_dx), _halve_rz(bi_d))
    rec_params = (vn_t, bhn_vn, d_t0, d_t1, bhn_d)
    out_params = (wo_v, wo_d, bo)
    return pre_params, rec_params, out_params


def kernel(x, wd, bd, wi_v, bi_v, wh_v, bhn_v, wi_nx, wi_nd, wi_nv, bi_n,
           wh_n, bhn_n, wi_dx, wi_dv, wi_dn, bi_d, wh_d, bhn_d, wo_v, wo_d, bo):
    pre_params, rec_params, out_params = _repack(
        wd, bd, wi_v, bi_v, wh_v, bhn_v, wi_nx, wi_nd, wi_nv, bi_n,
        wh_n, bhn_n, wi_dx, wi_dv, wi_dn, bi_d, wh_d, bhn_d, wo_v, wo_d, bo)
    return _forward(x, pre_params, rec_params, out_params, tq=1024)
